# Initial kernel scaffold; baseline (speedup 1.0000x reference)
import jax
import jax.numpy as jnp
from jax.experimental import pallas as pl


def kernel(x, r, t):
    raise NotImplementedError("write your pallas kernel here")


# speedup vs baseline: 1.0428x; 1.0428x over previous
import functools

import jax
import jax.numpy as jnp
from jax import lax
from jax.experimental import pallas as pl
from jax.experimental.pallas import tpu as pltpu


def _fd_body(xb_ref, x4t_ref, rv_ref, tv_ref, o_ref, *, dmodel):
    xb = xb_ref[...]
    x4t = x4t_ref[...]
    rv = rv_ref[...]
    tv = tv_ref[...]

    dot = jnp.dot(xb * (-2.0), x4t, preferred_element_type=jnp.float32,
                  precision=lax.Precision.HIGHEST)

    a = (jnp.sum(xb * xb, axis=1, keepdims=True)
         + 2e-6 * jnp.sum(xb, axis=1, keepdims=True))
    bc = (jnp.sum(x4t * x4t, axis=0, keepdims=True)
          - 2e-6 * jnp.sum(x4t, axis=0, keepdims=True)
          + dmodel * 1e-12)

    d2 = dot + a + bc
    dist = jnp.sqrt(jnp.maximum(d2, 0.0))
    f = 1.0 / (jnp.exp((dist - rv) * tv) + 1.0)

    w = f.shape[1]
    f1 = pltpu.roll(f, w - 1, 1)
    f2 = pltpu.roll(f, w - 2, 1)
    f3 = pltpu.roll(f, w - 3, 1)
    noedge = 1.0 - jnp.maximum(f1, jnp.maximum(f2, f3))

    lane = lax.broadcasted_iota(jnp.int32, f.shape, 1)
    o_ref[...] = jnp.where((lane & 3) == 0, noedge, f)


@jax.jit
def kernel(x, r, t):
    b, n, dmodel = x.shape
    x2 = x[0]
    x4t = jnp.broadcast_to(x2[:, None, :], (n, 4, dmodel))
    x4t = x4t.reshape(n * 4, dmodel).T
    rv = jnp.tile(jnp.concatenate([r[:1], r]), n)[None, :]
    tv = jnp.tile(jnp.concatenate([t[:1], t]), n)[None, :]

    bi = 64
    grid = (n // bi,)
    out = pl.pallas_call(
        functools.partial(_fd_body, dmodel=dmodel),
        grid=grid,
        in_specs=[
            pl.BlockSpec((bi, dmodel), lambda i: (i, 0)),
            pl.BlockSpec((dmodel, 4 * n), lambda i: (0, 0)),
            pl.BlockSpec((1, 4 * n), lambda i: (0, 0)),
            pl.BlockSpec((1, 4 * n), lambda i: (0, 0)),
        ],
        out_specs=pl.BlockSpec((bi, 4 * n), lambda i: (i, 0)),
        out_shape=jax.ShapeDtypeStruct((n, 4 * n), jnp.float32),
    )(x2, x4t, rv, tv)
    return out.reshape(b, n, n, 4)


# speedup vs baseline: 3.0425x; 2.9178x over previous
import functools

import jax
import jax.numpy as jnp
from jax import lax
from jax.experimental import pallas as pl
from jax.experimental.pallas import tpu as pltpu


def _fd_body(xb_ref, x4t_ref, rv_ref, tv_ref, o_ref, *, dmodel):
    xb = xb_ref[...]
    x4t = x4t_ref[...]
    rv = rv_ref[...]
    tv = tv_ref[...]

    dot = jnp.dot(xb * (-2.0), x4t, preferred_element_type=jnp.float32,
                  precision=lax.Precision.HIGHEST)

    a = (jnp.sum(xb * xb, axis=1, keepdims=True)
         + 2e-6 * jnp.sum(xb, axis=1, keepdims=True))
    bc = (jnp.sum(x4t * x4t, axis=0, keepdims=True)
          - 2e-6 * jnp.sum(x4t, axis=0, keepdims=True)
          + dmodel * 1e-12)

    d2 = dot + a + bc
    dist = jnp.sqrt(jnp.maximum(d2, 0.0))
    f = 1.0 / (jnp.exp((dist - rv) * tv) + 1.0)

    w = f.shape[1]
    f1 = pltpu.roll(f, w - 128, 1)
    f2 = pltpu.roll(f, w - 256, 1)
    f3 = pltpu.roll(f, w - 384, 1)
    noedge = 1.0 - jnp.maximum(f1, jnp.maximum(f2, f3))

    lane = lax.broadcasted_iota(jnp.int32, f.shape, 1)
    res = jnp.where(((lane >> 7) & 3) == 0, noedge, f)
    o_ref[...] = res.reshape(o_ref.shape)


@jax.jit
def kernel(x, r, t):
    b, n, dmodel = x.shape
    nt = n // 128
    x2 = x[0]
    x4 = jnp.broadcast_to(x2.reshape(nt, 1, 128, dmodel),
                          (nt, 4, 128, dmodel))
    x4t = x4.reshape(4 * n, dmodel).T
    rv = jnp.tile(jnp.repeat(jnp.concatenate([r[:1], r]), 128), nt)[None, :]
    tv = jnp.tile(jnp.repeat(jnp.concatenate([t[:1], t]), 128), nt)[None, :]

    bi = 64
    grid = (n // bi,)
    out = pl.pallas_call(
        functools.partial(_fd_body, dmodel=dmodel),
        grid=grid,
        in_specs=[
            pl.BlockSpec((bi, dmodel), lambda i: (i, 0)),
            pl.BlockSpec((dmodel, 4 * n), lambda i: (0, 0)),
            pl.BlockSpec((1, 4 * n), lambda i: (0, 0)),
            pl.BlockSpec((1, 4 * n), lambda i: (0, 0)),
        ],
        out_specs=pl.BlockSpec((bi, 4 * nt, 128), lambda i: (i, 0, 0)),
        out_shape=jax.ShapeDtypeStruct((n, 4 * nt, 128), jnp.float32),
    )(x2, x4t, rv, tv)
    out = out.reshape(n, nt, 4, 128).transpose(0, 1, 3, 2)
    return out.reshape(b, n, n, 4)


# speedup vs baseline: 6.5628x; 2.1570x over previous
import functools

import jax
import jax.numpy as jnp
from jax import lax
from jax.experimental import pallas as pl
from jax.experimental.pallas import tpu as pltpu


def _fd_body(r_ref, t_ref, xb_ref, x2t_ref, o_ref, *, dmodel):
    xb = xb_ref[...]
    x2t = x2t_ref[...]

    dot = jnp.dot(xb * (-2.0), x2t, preferred_element_type=jnp.float32,
                  precision=lax.Precision.HIGHEST)

    a = (jnp.sum(xb * xb, axis=1, keepdims=True)
         + 2e-6 * jnp.sum(xb, axis=1, keepdims=True))
    bc = (jnp.sum(x2t * x2t, axis=0, keepdims=True)
          - 2e-6 * jnp.sum(x2t, axis=0, keepdims=True)
          + dmodel * 1e-12)

    dist = jnp.sqrt(jnp.maximum(dot + a + bc, 0.0))

    fs = [1.0 / (jnp.exp((dist - r_ref[k]) * t_ref[k]) + 1.0) for k in range(3)]
    noedge = 1.0 - jnp.maximum(fs[0], jnp.maximum(fs[1], fs[2]))
    planes = [noedge] + fs

    n = dist.shape[1]
    pieces = [p[:, jt * 128:(jt + 1) * 128]
              for jt in range(n // 128) for p in planes]
    res = jnp.concatenate(pieces, axis=1)
    o_ref[...] = res.reshape(o_ref.shape)


@jax.jit
def kernel(x, r, t):
    b, n, dmodel = x.shape
    nt = n // 128
    x2 = x[0]
    x2t = x2.T

    bi = 64
    grid = (n // bi,)
    out = pl.pallas_call(
        functools.partial(_fd_body, dmodel=dmodel),
        grid=grid,
        in_specs=[
            pl.BlockSpec(memory_space=pltpu.SMEM),
            pl.BlockSpec(memory_space=pltpu.SMEM),
            pl.BlockSpec((bi, dmodel), lambda i: (i, 0)),
            pl.BlockSpec((dmodel, n), lambda i: (0, 0)),
        ],
        out_specs=pl.BlockSpec((bi, 4 * nt, 128), lambda i: (i, 0, 0)),
        out_shape=jax.ShapeDtypeStruct((n, 4 * nt, 128), jnp.float32),
    )(r, t, x2, x2t)
    out = out.reshape(n, nt, 4, 128).transpose(0, 1, 3, 2)
    return out.reshape(b, n, n, 4)


# speedup vs baseline: 7.0029x; 1.0671x over previous
import functools

import jax
import jax.numpy as jnp
from jax import lax
from jax.experimental import pallas as pl
from jax.experimental.pallas import tpu as pltpu


def _fd_body(r_ref, t_ref, xb_ref, x2t_ref, o_ref, *, dmodel):
    xb = xb_ref[...]
    x2t = x2t_ref[...]

    dot = jnp.dot(xb * (-2.0), x2t, preferred_element_type=jnp.float32,
                  precision=lax.Precision.HIGHEST)

    a = (jnp.sum(xb * xb, axis=1, keepdims=True)
         + 2e-6 * jnp.sum(xb, axis=1, keepdims=True))
    bc = (jnp.sum(x2t * x2t, axis=0, keepdims=True)
          - 2e-6 * jnp.sum(x2t, axis=0, keepdims=True)
          + dmodel * 1e-12)

    dist = jnp.sqrt(jnp.maximum(dot + a + bc, 0.0))

    fs = [1.0 / (jnp.exp((dist - r_ref[k]) * t_ref[k]) + 1.0) for k in range(3)]
    noedge = 1.0 - jnp.maximum(fs[0], jnp.maximum(fs[1], fs[2]))
    planes = [noedge] + fs

    n = dist.shape[1]
    pieces = [p[:, jt * 128:(jt + 1) * 128]
              for jt in range(n // 128) for p in planes]
    res = jnp.concatenate(pieces, axis=1)
    o_ref[...] = res.reshape(o_ref.shape)


@jax.jit
def kernel(x, r, t):
    b, n, dmodel = x.shape
    nt = n // 128
    x2 = x[0]
    x2t = x2.T

    bi = 128
    grid = (n // bi,)
    out = pl.pallas_call(
        functools.partial(_fd_body, dmodel=dmodel),
        grid=grid,
        in_specs=[
            pl.BlockSpec(memory_space=pltpu.SMEM),
            pl.BlockSpec(memory_space=pltpu.SMEM),
            pl.BlockSpec((bi, dmodel), lambda i: (i, 0)),
            pl.BlockSpec((dmodel, n), lambda i: (0, 0)),
        ],
        out_specs=pl.BlockSpec((bi, 4 * nt, 128), lambda i: (i, 0, 0)),
        out_shape=jax.ShapeDtypeStruct((n, 4 * nt, 128), jnp.float32),
    )(r, t, x2, x2t)
    out = out.reshape(n, nt, 4, 128).transpose(0, 1, 3, 2)
    return out.reshape(b, n, n, 4)
